# baseline (speedup 1.0000x reference)
import functools

import jax
import jax.numpy as jnp
from jax import lax
from jax.experimental import pallas as pl
from jax.experimental.pallas import tpu as pltpu
from jax.experimental.pallas import tpu_sc as plsc

N_FACTORS = 32
BATCH = 16384
NC = 2
NS = 16
NW = NC * NS
B_PER_W = BATCH // NW
CH = 128
NCHUNK = B_PER_W // CH


N_ROWS = 1000000
PER_TEC = B_PER_W * N_FACTORS
N_STREAMS = PER_TEC // CH


def _gather_body(uflat, mflat, users_hbm, movies_hbm, uout, mout,
                 uidx_v, midx_v, uoff_v, moff_v, urows_v, mrows_v, sem):
    wid = lax.axis_index("s") * NC + lax.axis_index("c")
    base = wid * B_PER_W
    pltpu.sync_copy(users_hbm.at[pl.ds(base, B_PER_W)], uidx_v)
    pltpu.sync_copy(movies_hbm.at[pl.ds(base, B_PER_W)], midx_v)

    def build_offsets(f, carry):
        for g in range(B_PER_W // 16):
            row = f * 4 + g // 8
            col = (g % 8) * 16
            uvec = uidx_v[pl.ds(g * 16, 16)]
            mvec = midx_v[pl.ds(g * 16, 16)]
            uoff_v[row, pl.ds(col, 16)] = uvec + f * N_ROWS
            moff_v[row, pl.ds(col, 16)] = mvec + f * N_ROWS
        return carry

    lax.fori_loop(0, N_FACTORS, build_offsets, 0)

    def fire(j, carry):
        pltpu.async_copy(uflat.at[uoff_v.at[j]],
                         urows_v.at[pl.ds(j * CH, CH)], sem)
        pltpu.async_copy(mflat.at[moff_v.at[j]],
                         mrows_v.at[pl.ds(j * CH, CH)], sem)
        return carry

    lax.fori_loop(0, N_STREAMS, fire, 0)
    pltpu.make_async_copy(uflat.at[pl.ds(0, PER_TEC)], urows_v, sem).wait()
    pltpu.make_async_copy(mflat.at[pl.ds(0, PER_TEC)], mrows_v, sem).wait()
    pltpu.sync_copy(urows_v, uout.at[wid])
    pltpu.sync_copy(mrows_v, mout.at[wid])


_gather_cache = []


def _gather(*args):
    if not _gather_cache:
        _gather_cache.append(functools.partial(
            pl.kernel,
            mesh=plsc.VectorSubcoreMesh(core_axis_name="c",
                                        subcore_axis_name="s"),
            out_type=[
                jax.ShapeDtypeStruct((NW, PER_TEC), jnp.float32),
                jax.ShapeDtypeStruct((NW, PER_TEC), jnp.float32),
            ],
            scratch_types=[
                pltpu.VMEM((B_PER_W,), jnp.int32),
                pltpu.VMEM((B_PER_W,), jnp.int32),
                pltpu.VMEM((N_STREAMS, CH), jnp.int32),
                pltpu.VMEM((N_STREAMS, CH), jnp.int32),
                pltpu.VMEM((PER_TEC,), jnp.float32),
                pltpu.VMEM((PER_TEC,), jnp.float32),
                pltpu.SemaphoreType.DMA,
            ],
        )(_gather_body))
    return _gather_cache[0](*args)


def _mlp_body(u_ref, m_ref, w1u_ref, w1m_ref, b1_ref, w2_ref, b2_ref,
              wf_ref, bf_ref, o_ref):
    x = jnp.dot(u_ref[...], w1u_ref[...], preferred_element_type=jnp.float32)
    x = x + jnp.dot(m_ref[...], w1m_ref[...], preferred_element_type=jnp.float32)
    h = jnp.maximum(x + b1_ref[...], 0.0)
    h = jnp.maximum(
        jnp.dot(h, w2_ref[...], preferred_element_type=jnp.float32)
        + b2_ref[...], 0.0)
    s = jnp.dot(h, wf_ref[...], preferred_element_type=jnp.float32) + bf_ref[...]
    o_ref[...] = jax.nn.sigmoid(s) * 4.5 + 0.5


def _mlp(u, m, w1u, w1m, b1, w2, b2, wf, bf, block_b=2048):
    nb = BATCH // block_b
    wspec = lambda shape: pl.BlockSpec(shape, lambda i: (0, 0))
    return pl.pallas_call(
        _mlp_body,
        grid=(nb,),
        in_specs=[
            pl.BlockSpec((block_b, N_FACTORS), lambda i: (i, 0)),
            pl.BlockSpec((block_b, N_FACTORS), lambda i: (i, 0)),
            wspec(w1u.shape),
            wspec(w1m.shape),
            wspec(b1.shape),
            wspec(w2.shape),
            wspec(b2.shape),
            wspec(wf.shape),
            wspec(bf.shape),
        ],
        out_specs=pl.BlockSpec((block_b, 1), lambda i: (i, 0)),
        out_shape=jax.ShapeDtypeStruct((BATCH, 1), jnp.float32),
    )(u, m, w1u, w1m, b1, w2, b2, wf, bf)


@jax.jit
def kernel(users, movies, user_emb, movie_emb, W1, b1, W2, b2, Wf, bf):
    u_raw, m_raw = _gather(user_emb.T.reshape(-1), movie_emb.T.reshape(-1),
                           users.astype(jnp.int32), movies.astype(jnp.int32))
    u_rows = u_raw.reshape(NW, N_FACTORS, B_PER_W).transpose(0, 2, 1)
    u_rows = u_rows.reshape(BATCH, N_FACTORS)
    m_rows = m_raw.reshape(NW, N_FACTORS, B_PER_W).transpose(0, 2, 1)
    m_rows = m_rows.reshape(BATCH, N_FACTORS)
    w1u = W1[:N_FACTORS]
    w1m = W1[N_FACTORS:]
    return _mlp(u_rows, m_rows, w1u, w1m,
                b1.reshape(1, -1), W2, b2.reshape(1, -1),
                Wf, bf.reshape(1, 1))


# speedup vs baseline: 7.4193x; 7.4193x over previous
import functools

import jax
import jax.numpy as jnp
from jax import lax
from jax.experimental import pallas as pl
from jax.experimental.pallas import tpu as pltpu
from jax.experimental.pallas import tpu_sc as plsc

N_FACTORS = 32
BATCH = 16384
N_ROWS = 1000000
NC = 2
NS = 16
NW = NC * NS
B_PER_W = BATCH // NW
PER_TEC = B_PER_W * N_FACTORS

DT_C = 4096
DT_G = DT_C // 128
DT_GRID = (N_ROWS + DT_C - 1) // DT_C
ROWS_LIN = DT_GRID * (DT_C // 4)


def _detile_body(x_ref, o_ref):
    y = x_ref[...].T
    for g in range(DT_G):
        parts = [y[128 * g + 32 * k: 128 * g + 32 * (k + 1), :]
                 for k in range(4)]
        o_ref[32 * g: 32 * (g + 1), :] = jnp.concatenate(parts, axis=1)


def _detile(xT):
    return pl.pallas_call(
        _detile_body,
        grid=(DT_GRID,),
        in_specs=[pl.BlockSpec((N_FACTORS, DT_C), lambda i: (0, i))],
        out_specs=pl.BlockSpec((DT_C // 4, 128), lambda i: (i, 0)),
        out_shape=jax.ShapeDtypeStruct((ROWS_LIN, 128), jnp.float32),
    )(xT)


def _gather_body(ulin, mlin, users_hbm, movies_hbm, uout, mout,
                 uidx_v, midx_v, ulane_v, mlane_v, rowidx_v, raw_v,
                 uex_v, mex_v, sem):
    wid = lax.axis_index("s") * NC + lax.axis_index("c")
    base = wid * B_PER_W
    pltpu.sync_copy(users_hbm.at[pl.ds(base, B_PER_W)], uidx_v)
    pltpu.sync_copy(movies_hbm.at[pl.ds(base, B_PER_W)], midx_v)

    iota = lax.iota(jnp.int32, 16)

    def prep(ids_v, lanes_v, table_sel):
        for g in range(B_PER_W // 16):
            vec = ids_v[pl.ds(g * 16, 16)]
            row = ((vec >> 7) << 5) + (vec & 31)
            rowidx_v[table_sel, g // 8, pl.ds((g % 8) * 16, 16)] = row
            lanes_v[pl.ds(g * 16, 16)] = ((vec >> 5) & 3) * 32

    prep(uidx_v, ulane_v, 0)
    prep(midx_v, mlane_v, 1)

    def gather_one(table_sel, lin, lanes_v, ex_v):
        copies = []
        for j in range(B_PER_W // 128):
            copies.append(pltpu.async_copy(
                lin.at[rowidx_v.at[table_sel, j]],
                raw_v.at[pl.ds(j * 128, 128)], sem))
        for c in copies:
            c.wait()

        def extract_f(f, carry):
            for g in range(B_PER_W // 16):
                rows16 = g * 16 + iota
                cols16 = lanes_v[pl.ds(g * 16, 16)] + f
                vals = plsc.load_gather(raw_v, [rows16, cols16])
                pos = rows16 * N_FACTORS + f
                plsc.store_scatter(ex_v, [pos], vals)
            return carry

        lax.fori_loop(0, N_FACTORS, extract_f, 0)

    gather_one(0, ulin, ulane_v, uex_v)
    gather_one(1, mlin, mlane_v, mex_v)
    pltpu.sync_copy(uex_v, uout.at[wid])
    pltpu.sync_copy(mex_v, mout.at[wid])


_gather_cache = []


def _gather(*args):
    if not _gather_cache:
        _gather_cache.append(functools.partial(
            pl.kernel,
            mesh=plsc.VectorSubcoreMesh(core_axis_name="c",
                                        subcore_axis_name="s"),
            out_type=[
                jax.ShapeDtypeStruct((NW, PER_TEC), jnp.float32),
                jax.ShapeDtypeStruct((NW, PER_TEC), jnp.float32),
            ],
            scratch_types=[
                pltpu.VMEM((B_PER_W,), jnp.int32),
                pltpu.VMEM((B_PER_W,), jnp.int32),
                pltpu.VMEM((B_PER_W,), jnp.int32),
                pltpu.VMEM((B_PER_W,), jnp.int32),
                pltpu.VMEM((2, B_PER_W // 128, 128), jnp.int32),
                pltpu.VMEM((B_PER_W, 128), jnp.float32),
                pltpu.VMEM((PER_TEC,), jnp.float32),
                pltpu.VMEM((PER_TEC,), jnp.float32),
                pltpu.SemaphoreType.DMA,
            ],
            compiler_params=pltpu.CompilerParams(needs_layout_passes=False),
        )(_gather_body))
    return _gather_cache[0](*args)


def _mlp_body(u_ref, m_ref, w1u_ref, w1m_ref, b1_ref, w2_ref, b2_ref,
              wf_ref, bf_ref, o_ref):
    x = jnp.dot(u_ref[...], w1u_ref[...], preferred_element_type=jnp.float32)
    x = x + jnp.dot(m_ref[...], w1m_ref[...], preferred_element_type=jnp.float32)
    h = jnp.maximum(x + b1_ref[...], 0.0)
    h = jnp.maximum(
        jnp.dot(h, w2_ref[...], preferred_element_type=jnp.float32)
        + b2_ref[...], 0.0)
    s = jnp.dot(h, wf_ref[...], preferred_element_type=jnp.float32) + bf_ref[...]
    o_ref[...] = jax.nn.sigmoid(s) * 4.5 + 0.5


def _mlp(u, m, w1u, w1m, b1, w2, b2, wf, bf, block_b=2048):
    nb = BATCH // block_b
    wspec = lambda shape: pl.BlockSpec(shape, lambda i: (0, 0))
    return pl.pallas_call(
        _mlp_body,
        grid=(nb,),
        in_specs=[
            pl.BlockSpec((block_b, N_FACTORS), lambda i: (i, 0)),
            pl.BlockSpec((block_b, N_FACTORS), lambda i: (i, 0)),
            wspec(w1u.shape),
            wspec(w1m.shape),
            wspec(b1.shape),
            wspec(w2.shape),
            wspec(b2.shape),
            wspec(wf.shape),
            wspec(bf.shape),
        ],
        out_specs=pl.BlockSpec((block_b, 1), lambda i: (i, 0)),
        out_shape=jax.ShapeDtypeStruct((BATCH, 1), jnp.float32),
    )(u, m, w1u, w1m, b1, w2, b2, wf, bf)


@jax.jit
def kernel(users, movies, user_emb, movie_emb, W1, b1, W2, b2, Wf, bf):
    u_lin = _detile(user_emb.T)
    m_lin = _detile(movie_emb.T)
    u_raw, m_raw = _gather(u_lin, m_lin,
                           users.astype(jnp.int32), movies.astype(jnp.int32))
    u_rows = u_raw.reshape(BATCH, N_FACTORS)
    m_rows = m_raw.reshape(BATCH, N_FACTORS)
    w1u = W1[:N_FACTORS]
    w1m = W1[N_FACTORS:]
    return _mlp(u_rows, m_rows, w1u, w1m,
                b1.reshape(1, -1), W2, b2.reshape(1, -1),
                Wf, bf.reshape(1, 1))


# speedup vs baseline: 7.4424x; 1.0031x over previous
import functools

import jax
import jax.numpy as jnp
from jax import lax
from jax.experimental import pallas as pl
from jax.experimental.pallas import tpu as pltpu
from jax.experimental.pallas import tpu_sc as plsc

N_FACTORS = 32
BATCH = 16384
N_ROWS = 1000000
NC = 2
NS = 16
NW = NC * NS
B_PER_W = BATCH // NW
PER_TEC = B_PER_W * N_FACTORS

DT_C = 4096
DT_G = DT_C // 128
DT_GRID = (N_ROWS + DT_C - 1) // DT_C
ROWS_LIN = DT_GRID * (DT_C // 4)


def _detile_body(x_ref, o_ref):
    for q in range(4):
        xq = x_ref[:, 1024 * q: 1024 * (q + 1)]
        o_ref[:, 32 * q: 32 * (q + 1)] = xq.T


def _detile(xT):
    return pl.pallas_call(
        _detile_body,
        grid=(DT_GRID,),
        in_specs=[pl.BlockSpec((N_FACTORS, DT_C), lambda i: (0, i))],
        out_specs=pl.BlockSpec((DT_C // 4, 128), lambda i: (i, 0)),
        out_shape=jax.ShapeDtypeStruct((ROWS_LIN, 128), jnp.float32),
    )(xT)


def _gather_body(ulin, mlin, users_hbm, movies_hbm, uout, mout,
                 uidx_v, midx_v, ulane_v, mlane_v, rowidx_v, raw_v,
                 uex_v, mex_v, sem):
    wid = lax.axis_index("s") * NC + lax.axis_index("c")
    base = wid * B_PER_W
    pltpu.sync_copy(users_hbm.at[pl.ds(base, B_PER_W)], uidx_v)
    pltpu.sync_copy(movies_hbm.at[pl.ds(base, B_PER_W)], midx_v)

    iota = lax.iota(jnp.int32, 16)

    def prep(ids_v, lanes_v, table_sel):
        for g in range(B_PER_W // 16):
            vec = ids_v[pl.ds(g * 16, 16)]
            row = ((vec >> 12) << 10) + (vec & 1023)
            rowidx_v[table_sel, g // 8, pl.ds((g % 8) * 16, 16)] = row
            lanes_v[pl.ds(g * 16, 16)] = ((vec >> 10) & 3) * 32

    prep(uidx_v, ulane_v, 0)
    prep(midx_v, mlane_v, 1)

    def gather_one(table_sel, lin, lanes_v, ex_v):
        copies = []
        for j in range(B_PER_W // 128):
            copies.append(pltpu.async_copy(
                lin.at[rowidx_v.at[table_sel, j]],
                raw_v.at[pl.ds(j * 128, 128)], sem))
        for c in copies:
            c.wait()

        def extract_f(f, carry):
            for g in range(B_PER_W // 16):
                rows16 = g * 16 + iota
                cols16 = lanes_v[pl.ds(g * 16, 16)] + f
                vals = plsc.load_gather(raw_v, [rows16, cols16])
                pos = rows16 * N_FACTORS + f
                plsc.store_scatter(ex_v, [pos], vals)
            return carry

        lax.fori_loop(0, N_FACTORS, extract_f, 0)

    gather_one(0, ulin, ulane_v, uex_v)
    gather_one(1, mlin, mlane_v, mex_v)
    pltpu.sync_copy(uex_v, uout.at[wid])
    pltpu.sync_copy(mex_v, mout.at[wid])


_gather_cache = []


def _gather(*args):
    if not _gather_cache:
        _gather_cache.append(functools.partial(
            pl.kernel,
            mesh=plsc.VectorSubcoreMesh(core_axis_name="c",
                                        subcore_axis_name="s"),
            out_type=[
                jax.ShapeDtypeStruct((NW, PER_TEC), jnp.float32),
                jax.ShapeDtypeStruct((NW, PER_TEC), jnp.float32),
            ],
            scratch_types=[
                pltpu.VMEM((B_PER_W,), jnp.int32),
                pltpu.VMEM((B_PER_W,), jnp.int32),
                pltpu.VMEM((B_PER_W,), jnp.int32),
                pltpu.VMEM((B_PER_W,), jnp.int32),
                pltpu.VMEM((2, B_PER_W // 128, 128), jnp.int32),
                pltpu.VMEM((B_PER_W, 128), jnp.float32),
                pltpu.VMEM((PER_TEC,), jnp.float32),
                pltpu.VMEM((PER_TEC,), jnp.float32),
                pltpu.SemaphoreType.DMA,
            ],
            compiler_params=pltpu.CompilerParams(needs_layout_passes=False),
        )(_gather_body))
    return _gather_cache[0](*args)


def _mlp_body(u_ref, m_ref, w1u_ref, w1m_ref, b1_ref, w2_ref, b2_ref,
              wf_ref, bf_ref, o_ref):
    x = jnp.dot(u_ref[...], w1u_ref[...], preferred_element_type=jnp.float32)
    x = x + jnp.dot(m_ref[...], w1m_ref[...], preferred_element_type=jnp.float32)
    h = jnp.maximum(x + b1_ref[...], 0.0)
    h = jnp.maximum(
        jnp.dot(h, w2_ref[...], preferred_element_type=jnp.float32)
        + b2_ref[...], 0.0)
    s = jnp.dot(h, wf_ref[...], preferred_element_type=jnp.float32) + bf_ref[...]
    o_ref[...] = jax.nn.sigmoid(s) * 4.5 + 0.5


def _mlp(u, m, w1u, w1m, b1, w2, b2, wf, bf, block_b=2048):
    nb = BATCH // block_b
    wspec = lambda shape: pl.BlockSpec(shape, lambda i: (0, 0))
    return pl.pallas_call(
        _mlp_body,
        grid=(nb,),
        in_specs=[
            pl.BlockSpec((block_b, N_FACTORS), lambda i: (i, 0)),
            pl.BlockSpec((block_b, N_FACTORS), lambda i: (i, 0)),
            wspec(w1u.shape),
            wspec(w1m.shape),
            wspec(b1.shape),
            wspec(w2.shape),
            wspec(b2.shape),
            wspec(wf.shape),
            wspec(bf.shape),
        ],
        out_specs=pl.BlockSpec((block_b, 1), lambda i: (i, 0)),
        out_shape=jax.ShapeDtypeStruct((BATCH, 1), jnp.float32),
    )(u, m, w1u, w1m, b1, w2, b2, wf, bf)


@jax.jit
def kernel(users, movies, user_emb, movie_emb, W1, b1, W2, b2, Wf, bf):
    u_lin = _detile(user_emb.T)
    m_lin = _detile(movie_emb.T)
    u_raw, m_raw = _gather(u_lin, m_lin,
                           users.astype(jnp.int32), movies.astype(jnp.int32))
    u_rows = u_raw.reshape(BATCH, N_FACTORS)
    m_rows = m_raw.reshape(BATCH, N_FACTORS)
    w1u = W1[:N_FACTORS]
    w1m = W1[N_FACTORS:]
    return _mlp(u_rows, m_rows, w1u, w1m,
                b1.reshape(1, -1), W2, b2.reshape(1, -1),
                Wf, bf.reshape(1, 1))
